# baseline (speedup 1.0000x reference)
import jax
import jax.numpy as jnp
from jax import lax
from jax.experimental import pallas as pl
from jax.experimental.pallas import tpu as pltpu
from jax.experimental.pallas import tpu_sc as plsc

SCALE_FACTOR = 0.9
ZERO_POINT = 1e-08

BATCH = 128
DIM = 64
NOUT = 8192
MULT = NOUT // BATCH
PROW = DIM * MULT
NW = 32
ROWS_PER_W = BATCH // NW
L = 16


def _reduce_lanes(vec, op):
    s = vec[0]
    for i in range(1, L):
        s = op(s, vec[i])
    return s


def _sc_fused_body(zm_hbm, zv_hbm, x_hbm, eps_hbm, out_hbm,
                   zm_v, zv_v, eps_v, p_v, cnt_v, x_v, o_v,
                   in_sem0, in_sem1, out_sem0, out_sem1):
    wid = lax.axis_index("s") * 2 + lax.axis_index("c")
    r0 = wid * ROWS_PER_W
    ones = jnp.ones((L,), jnp.float32)
    in_sems = [in_sem0, in_sem1]
    out_sems = [out_sem0, out_sem1]

    pltpu.sync_copy(zm_hbm.at[pl.ds(r0, ROWS_PER_W)], zm_v)
    pltpu.sync_copy(zv_hbm.at[pl.ds(r0, ROWS_PER_W)], zv_v)

    def start_in(rl):
        b = rl % 2
        h1 = pltpu.async_copy(
            eps_hbm.at[pl.ds((r0 + rl) * MULT, MULT)], eps_v.at[b], in_sems[b])
        h2 = pltpu.async_copy(
            x_hbm.at[pl.ds(r0 + rl, 1)], x_v.at[pl.ds(b, 1)], in_sems[b])
        return (h1, h2)

    in_handles = {0: start_in(0)}
    out_handles = {}

    for rl in range(ROWS_PER_W):
        b = rl % 2
        if rl + 1 < ROWS_PER_W:
            in_handles[rl + 1] = start_in(rl + 1)

        ZU = 16
        zero = jnp.zeros((L,), jnp.float32)

        @plsc.parallel_loop(0, NOUT, step=ZU * L)
        def _(j):
            for k in range(ZU):
                cnt_v[pl.ds(j + k * L, L)] = zero

        if rl - 2 in out_handles:
            for h in out_handles.pop(rl - 2):
                h.wait()
        for h in in_handles.pop(rl):
            h.wait()

        scs = [jnp.exp(0.5 * zv_v[rl, pl.ds(k * L, L)]) for k in range(DIM // L)]
        zms = [zm_v[rl, pl.ds(k * L, L)] for k in range(DIM // L)]

        big = jnp.full((L,), jnp.inf, jnp.float32)
        KU = DIM // L

        @plsc.parallel_loop(0, MULT, step=1, unroll=2,
                            carry=((big,) * KU, (-big,) * KU))
        def samp_out(e, carry):
            mins, maxs = carry
            nmins, nmaxs = [], []
            for k in range(KU):
                v = zms[k] + scs[k] * eps_v[b, e, pl.ds(k * L, L)]
                p_v[pl.ds(e * DIM + k * L, L)] = v
                nmins.append(jnp.minimum(mins[k], v))
                nmaxs.append(jnp.maximum(maxs[k], v))
            return tuple(nmins), tuple(nmaxs)

        mins, maxs = samp_out
        vmin_v = jnp.minimum(jnp.minimum(mins[0], mins[1]), jnp.minimum(mins[2], mins[3]))
        vmax_v = jnp.maximum(jnp.maximum(maxs[0], maxs[1]), jnp.maximum(maxs[2], maxs[3]))
        vmin = _reduce_lanes(vmin_v, jnp.minimum)
        vmax = _reduce_lanes(vmax_v, jnp.maximum)
        width = (vmax - vmin) * jnp.float32(1.0 / NOUT)
        width = jnp.where(width <= 0.0, jnp.float32(1.0), width)
        vmin_b = jnp.full((L,), vmin, jnp.float32)
        inv_w_b = jnp.float32(1.0) / jnp.full((L,), width, jnp.float32)

        SU = 16

        @plsc.parallel_loop(0, PROW, step=SU * L)
        def _(j):
            for k in range(SU):
                v = p_v[pl.ds(j + k * L, L)]
                idx = ((v - vmin_b) * inv_w_b).astype(jnp.int32)
                idx = jnp.minimum(idx, NOUT - 1)
                plsc.addupdate_scatter(cnt_v, [idx], ones)

        MU = 8
        nbig = jnp.full((L,), -jnp.inf, jnp.float32)

        @plsc.parallel_loop(0, NOUT, step=MU * L, carry=(nbig,) * MU)
        def max_out(j, ms):
            return tuple(
                jnp.maximum(ms[k], cnt_v[pl.ds(j + k * L, L)])
                for k in range(MU)
            )

        ms = max_out
        maxc_v = ms[0]
        for k in range(1, MU):
            maxc_v = jnp.maximum(maxc_v, ms[k])
        maxc_b = jnp.full((L,), _reduce_lanes(maxc_v, jnp.maximum), jnp.float32)

        zero16 = jnp.zeros((L,), jnp.float32)

        @plsc.parallel_loop(0, NOUT, step=MU * L, carry=(zero16,) * MU)
        def sum_out(j, accs):
            naccs = []
            for k in range(MU):
                e = jnp.exp(cnt_v[pl.ds(j + k * L, L)] - maxc_b)
                cnt_v[pl.ds(j + k * L, L)] = e
                naccs.append(accs[k] + e)
            return tuple(naccs)

        accs = sum_out
        ssum = accs[0]
        for k in range(1, MU):
            ssum = ssum + accs[k]
        z_total = _reduce_lanes(ssum, lambda a, b: a + b)
        rz_b = jnp.float32(1.0) / jnp.full((L,), z_total, jnp.float32)
        thr_b = jnp.full((L,), jnp.float32(ZERO_POINT), jnp.float32)
        inv_s = jnp.float32(1.0 / SCALE_FACTOR)

        FU = 8

        @plsc.parallel_loop(0, NOUT, step=FU * L)
        def _(j):
            for k in range(FU):
                e = cnt_v[pl.ds(j + k * L, L)]
                p = e * rz_b
                o = x_v[b, pl.ds(j + k * L, L)] * p * inv_s
                o_v[b, pl.ds(j + k * L, L)] = jnp.where(p < thr_b, zero16, o)

        out_handles[rl] = (pltpu.async_copy(
            o_v.at[pl.ds(b, 1)], out_hbm.at[pl.ds(r0 + rl, 1)], out_sems[b]),)

    for rl in sorted(out_handles):
        for h in out_handles[rl]:
            h.wait()


def kernel(z_mean, z_var, x, epsilon):
    mesh = plsc.VectorSubcoreMesh(core_axis_name="c", subcore_axis_name="s")
    f = pl.kernel(
        _sc_fused_body,
        mesh=mesh,
        out_type=jax.ShapeDtypeStruct((BATCH, NOUT), jnp.float32),
        scratch_types=[
            pltpu.VMEM((ROWS_PER_W, DIM), jnp.float32),
            pltpu.VMEM((ROWS_PER_W, DIM), jnp.float32),
            pltpu.VMEM((2, MULT, DIM), jnp.float32),
            pltpu.VMEM((PROW,), jnp.float32),
            pltpu.VMEM((NOUT,), jnp.float32),
            pltpu.VMEM((2, NOUT), jnp.float32),
            pltpu.VMEM((2, NOUT), jnp.float32),
            pltpu.SemaphoreType.DMA,
            pltpu.SemaphoreType.DMA,
            pltpu.SemaphoreType.DMA,
            pltpu.SemaphoreType.DMA,
        ],
        compiler_params=pltpu.CompilerParams(needs_layout_passes=False),
    )
    return f(z_mean, z_var, x, epsilon)


# speedup vs baseline: 1.1390x; 1.1390x over previous
import jax
import jax.numpy as jnp
from jax import lax
from jax.experimental import pallas as pl
from jax.experimental.pallas import tpu as pltpu
from jax.experimental.pallas import tpu_sc as plsc

SCALE_FACTOR = 0.9
ZERO_POINT = 1e-08

BATCH = 128
DIM = 64
NOUT = 8192
MULT = NOUT // BATCH
PROW = DIM * MULT
NW = 32
ROWS_PER_W = BATCH // NW
L = 16


def _reduce_lanes(vec, op):
    s = vec[0]
    for i in range(1, L):
        s = op(s, vec[i])
    return s


def _sc_fused_body(zm_hbm, zv_hbm, x_hbm, eps_hbm, out_hbm,
                   zm_v, zv_v, eps_v, p_v, cnt_v, x_v, o_v,
                   in_sem, out_sem):
    wid = lax.axis_index("s") * 2 + lax.axis_index("c")
    r0 = wid * ROWS_PER_W
    ones = jnp.ones((L,), jnp.float32)

    pltpu.sync_copy(zm_hbm.at[pl.ds(r0, ROWS_PER_W)], zm_v)
    pltpu.sync_copy(zv_hbm.at[pl.ds(r0, ROWS_PER_W)], zv_v)

    def in_copies(rl):
        b = lax.rem(rl, 2)
        h1 = pltpu.make_async_copy(
            eps_hbm.at[pl.ds((r0 + rl) * MULT, MULT)], eps_v.at[b], in_sem)
        h2 = pltpu.make_async_copy(
            x_hbm.at[pl.ds(r0 + rl, 1)], x_v.at[pl.ds(b, 1)], in_sem)
        return h1, h2

    def out_copy(rl):
        b = lax.rem(rl, 2)
        return pltpu.make_async_copy(
            o_v.at[pl.ds(b, 1)], out_hbm.at[pl.ds(r0 + rl, 1)], out_sem)

    for h in in_copies(jnp.int32(0)):
        h.start()

    def row_body(rl, _):
        b = lax.rem(rl, 2)

        @pl.when(rl < ROWS_PER_W - 1)
        def _():
            for h in in_copies(rl + 1):
                h.start()

        ZU = 16
        zero = jnp.zeros((L,), jnp.float32)

        @plsc.parallel_loop(0, NOUT, step=ZU * L)
        def _(j):
            for k in range(ZU):
                cnt_v[pl.ds(j + k * L, L)] = zero

        @pl.when(rl >= 2)
        def _():
            out_copy(rl - 2).wait()

        for h in in_copies(rl):
            h.wait()

        scs = [jnp.exp(0.5 * zv_v[rl, pl.ds(k * L, L)]) for k in range(DIM // L)]
        zms = [zm_v[rl, pl.ds(k * L, L)] for k in range(DIM // L)]

        big = jnp.full((L,), jnp.inf, jnp.float32)
        KU = DIM // L

        @plsc.parallel_loop(0, MULT, step=1, unroll=4,
                            carry=((big,) * KU, (-big,) * KU))
        def samp_out(e, carry):
            mins, maxs = carry
            nmins, nmaxs = [], []
            for k in range(KU):
                v = zms[k] + scs[k] * eps_v[b, e, pl.ds(k * L, L)]
                p_v[pl.ds(e * DIM + k * L, L)] = v
                nmins.append(jnp.minimum(mins[k], v))
                nmaxs.append(jnp.maximum(maxs[k], v))
            return tuple(nmins), tuple(nmaxs)

        mins, maxs = samp_out
        vmin_v = jnp.minimum(jnp.minimum(mins[0], mins[1]), jnp.minimum(mins[2], mins[3]))
        vmax_v = jnp.maximum(jnp.maximum(maxs[0], maxs[1]), jnp.maximum(maxs[2], maxs[3]))
        vmin = _reduce_lanes(vmin_v, jnp.minimum)
        vmax = _reduce_lanes(vmax_v, jnp.maximum)
        width = (vmax - vmin) * jnp.float32(1.0 / NOUT)
        width = jnp.where(width <= 0.0, jnp.float32(1.0), width)
        vmin_b = jnp.full((L,), vmin, jnp.float32)
        inv_w_b = jnp.float32(1.0) / jnp.full((L,), width, jnp.float32)

        SU = 16

        @plsc.parallel_loop(0, PROW, step=SU * L)
        def _(j):
            for k in range(SU):
                v = p_v[pl.ds(j + k * L, L)]
                idx = ((v - vmin_b) * inv_w_b).astype(jnp.int32)
                idx = jnp.minimum(idx, NOUT - 1)
                plsc.addupdate_scatter(cnt_v, [idx], ones)

        MU = 8
        nbig = jnp.full((L,), -jnp.inf, jnp.float32)

        @plsc.parallel_loop(0, NOUT, step=MU * L, carry=(nbig,) * MU)
        def max_out(j, ms):
            return tuple(
                jnp.maximum(ms[k], cnt_v[pl.ds(j + k * L, L)])
                for k in range(MU)
            )

        ms = max_out
        maxc_v = ms[0]
        for k in range(1, MU):
            maxc_v = jnp.maximum(maxc_v, ms[k])
        maxc_b = jnp.full((L,), _reduce_lanes(maxc_v, jnp.maximum), jnp.float32)

        zero16 = jnp.zeros((L,), jnp.float32)

        @plsc.parallel_loop(0, NOUT, step=MU * L, carry=(zero16,) * MU)
        def sum_out(j, accs):
            naccs = []
            for k in range(MU):
                e = jnp.exp(cnt_v[pl.ds(j + k * L, L)] - maxc_b)
                cnt_v[pl.ds(j + k * L, L)] = e
                naccs.append(accs[k] + e)
            return tuple(naccs)

        accs = sum_out
        ssum = accs[0]
        for k in range(1, MU):
            ssum = ssum + accs[k]
        z_total = _reduce_lanes(ssum, lambda a, b: a + b)
        rz_b = jnp.float32(1.0) / jnp.full((L,), z_total, jnp.float32)
        thr_b = jnp.full((L,), jnp.float32(ZERO_POINT), jnp.float32)
        inv_s = jnp.float32(1.0 / SCALE_FACTOR)

        FU = 8

        @plsc.parallel_loop(0, NOUT, step=FU * L)
        def _(j):
            for k in range(FU):
                e = cnt_v[pl.ds(j + k * L, L)]
                p = e * rz_b
                o = x_v[b, pl.ds(j + k * L, L)] * p * inv_s
                o_v[b, pl.ds(j + k * L, L)] = jnp.where(p < thr_b, zero16, o)

        out_copy(rl).start()
        return 0

    lax.fori_loop(0, ROWS_PER_W, row_body, 0)

    out_copy(jnp.int32(ROWS_PER_W - 2)).wait()
    out_copy(jnp.int32(ROWS_PER_W - 1)).wait()


def kernel(z_mean, z_var, x, epsilon):
    mesh = plsc.VectorSubcoreMesh(core_axis_name="c", subcore_axis_name="s")
    f = pl.kernel(
        _sc_fused_body,
        mesh=mesh,
        out_type=jax.ShapeDtypeStruct((BATCH, NOUT), jnp.float32),
        scratch_types=[
            pltpu.VMEM((ROWS_PER_W, DIM), jnp.float32),
            pltpu.VMEM((ROWS_PER_W, DIM), jnp.float32),
            pltpu.VMEM((2, MULT, DIM), jnp.float32),
            pltpu.VMEM((PROW,), jnp.float32),
            pltpu.VMEM((NOUT,), jnp.float32),
            pltpu.VMEM((2, NOUT), jnp.float32),
            pltpu.VMEM((2, NOUT), jnp.float32),
            pltpu.SemaphoreType.DMA,
            pltpu.SemaphoreType.DMA,
        ],
        compiler_params=pltpu.CompilerParams(needs_layout_passes=False),
    )
    return f(z_mean, z_var, x, epsilon)
